# baseline (speedup 1.0000x reference)
import functools

import jax
import jax.numpy as jnp
from jax import lax
from jax.experimental import pallas as pl
from jax.experimental.pallas import tpu as pltpu
from jax.experimental.pallas import tpu_sc as plsc

N = 10000
E = 320000
NC, NS = 2, 16
NW = NC * NS
RPT = 320
NPAD = NW * RPT
CAP = 16384
CAP2 = CAP // 2
HALF = N // 2
WF = 16000
W = 64

_MESH = plsc.VectorSubcoreMesh(core_axis_name="c", subcore_axis_name="s")


def _wid():
    return lax.axis_index("s") * NC + lax.axis_index("c")


def _partition_body(src_hbm, dst_hbm, ew_hbm, srcp, dstlp, ewp, nwin,
                    srcw, dstw, eww, srcl, dstl, ewl, nv):
    wid = _wid()
    lo = wid * RPT
    hi = jnp.minimum(lo + RPT, N)
    pad_src = lax.rem(lo, HALF)

    def init_b(i, carry):
        s = pl.ds(i * 16, 16)
        srcl[s] = jnp.full((16,), pad_src, jnp.int32)
        dstl[s] = jnp.full((16,), RPT, jnp.int32)
        ewl[s] = jnp.zeros((16,), jnp.float32)
        return carry
    lax.fori_loop(0, CAP // 16, init_b, 0)

    def win_b(g, ptrs):
        base = g * WF
        pltpu.sync_copy(src_hbm.at[pl.ds(base, WF)], srcw)
        pltpu.sync_copy(dst_hbm.at[pl.ds(base, WF)], dstw)
        pltpu.sync_copy(ew_hbm.at[pl.ds(base, WF)], eww)

        def vec_b(j, ptrs):
            p0, p1 = ptrs
            s = pl.ds(j * 16, 16)
            dv = dstw[s]
            sv = srcw[s]
            wv = eww[s]
            m = (dv >= lo) & (dv < hi)
            lane = lax.iota(jnp.int32, 16)
            outp = []
            for mk, pk, rbase, shift in ((m & (sv < HALF), p0, 0, 0),
                                         (m & (sv >= HALF), p1, CAP2, HALF)):
                x = jnp.where(mk, 1, 0)
                for k in (1, 2, 4, 8):
                    sh = x.at[jnp.maximum(lane - k, 0)].get(
                        mode="promise_in_bounds")
                    x = x + jnp.where(lane >= k, sh, 0)
                cnt = x[15]
                tgt = lane + 1
                b = jnp.zeros((16,), jnp.int32)
                for k in (8, 4, 2, 1):
                    pv = x.at[b + (k - 1)].get(mode="promise_in_bounds")
                    b = b + jnp.where(pv < tgt, k, 0)
                perm = jnp.minimum(b, 15)
                pc = jnp.minimum(pk, CAP2 - 16)

                def tk(v, perm=perm):
                    return v.at[perm].get(mode="promise_in_bounds")
                dstl[pl.ds(rbase + pc, 16)] = tk(dv - lo)
                srcl[pl.ds(rbase + pc, 16)] = tk(sv - shift)
                ewl[pl.ds(rbase + pc, 16)] = tk(wv)
                outp.append(pc + cnt)
            return tuple(outp)
        return lax.fori_loop(0, WF // 16, vec_b, ptrs)

    cnt0, cnt1 = lax.fori_loop(0, E // WF, win_b,
                               (jnp.int32(0), jnp.int32(0)))
    for ck, rbase in ((cnt0, 0), (cnt1, CAP2)):
        pt = rbase + jnp.minimum(ck, CAP2 - 16)
        srcl[pl.ds(pt, 16)] = jnp.full((16,), pad_src, jnp.int32)
        dstl[pl.ds(pt, 16)] = jnp.full((16,), RPT, jnp.int32)
        ewl[pl.ds(pt, 16)] = jnp.zeros((16,), jnp.float32)
    lane = lax.iota(jnp.int32, 16)
    nv[...] = jnp.where(lane < 8, cnt0, cnt1)
    pltpu.sync_copy(nv, nwin.at[pl.ds(wid * 16, 16)])
    pltpu.sync_copy(srcl.at[pl.ds(0, CAP)], srcp.at[pl.ds(wid * CAP, CAP)])
    pltpu.sync_copy(dstl.at[pl.ds(0, CAP)], dstlp.at[pl.ds(wid * CAP, CAP)])
    pltpu.sync_copy(ewl.at[pl.ds(0, CAP)], ewp.at[pl.ds(wid * CAP, CAP)])


def _partition(src, dst, ew):
    f = pl.kernel(
        _partition_body,
        out_type=[
            jax.ShapeDtypeStruct((NW * CAP,), jnp.int32),
            jax.ShapeDtypeStruct((NW * CAP,), jnp.int32),
            jax.ShapeDtypeStruct((NW * CAP,), jnp.float32),
            jax.ShapeDtypeStruct((NW * 16,), jnp.int32),
        ],
        mesh=_MESH,
        scratch_types=[
            pltpu.VMEM((WF,), jnp.int32),
            pltpu.VMEM((WF,), jnp.int32),
            pltpu.VMEM((WF,), jnp.float32),
            pltpu.VMEM((CAP + 16,), jnp.int32),
            pltpu.VMEM((CAP + 16,), jnp.int32),
            pltpu.VMEM((CAP + 16,), jnp.float32),
            pltpu.VMEM((16,), jnp.int32),
        ],
    )
    return f(src, dst, ew)


def _segmax_body(h_hbm, srcp, dstlp, ewp, nwin, out_hbm,
                 a0, a1, a2, a3, a4, a5, a6, a7,
                 idxc, dstc, ewc, rows, nv, hsh, sems):
    wid = _wid()
    sid = lax.axis_index("s")
    w = 128
    chunk = 2048
    wpc = chunk // w
    accs = (a0, a1, a2, a3, a4, a5, a6, a7)

    def z_r(r, carry):
        for acc in accs:
            acc[pl.ds(r * 16, 16)] = jnp.zeros((16,), jnp.float32)
        return carry
    lax.fori_loop(0, RPT + 1, z_r, 0)

    pltpu.sync_copy(nwin.at[pl.ds(wid * 16, 16)], nv)
    nvv = nv[...]

    for half, lane0 in ((0, 0), (1, 8)):
        nw = lax.div(nvv[lane0] + (w - 1), w)
        lbase = wid * CAP + half * CAP2
        hrow = half * HALF

        pltpu.sync_copy(h_hbm.at[pl.ds(hrow + sid * 312, 312)],
                        hsh.at[pl.ds(sid * 312, 312)])

        @pl.when(sid == 0)
        def _tail(hrow=hrow):
            pltpu.sync_copy(h_hbm.at[pl.ds(hrow + 4992, 8)],
                            hsh.at[pl.ds(4992, 8)])
        plsc.subcore_barrier()

        def step(g, carry, nw=nw, lbase=lbase):
            sel_c = lax.rem(lax.div(g, wpc), 2)
            off = lax.rem(g, wpc) * w

            @pl.when(g < nw)
            def _issue():
                @pl.when(lax.rem(g, wpc) == 0)
                def _load_chunk():
                    cb = lbase + lax.div(g, wpc) * chunk
                    pltpu.sync_copy(srcp.at[pl.ds(cb, chunk)],
                                    idxc.at[sel_c])
                    pltpu.sync_copy(dstlp.at[pl.ds(cb, chunk)],
                                    dstc.at[sel_c])
                    pltpu.sync_copy(ewp.at[pl.ds(cb, chunk)], ewc.at[sel_c])
                pltpu.async_copy(hsh.at[idxc.at[sel_c, pl.ds(off, w)]],
                                 rows.at[lax.rem(g, 2)],
                                 sems.at[lax.rem(g, 2)])

            @pl.when(g > 0)
            def _compute():
                gp = g - 1
                sel_p = lax.rem(lax.div(gp, wpc), 2)
                offp = lax.rem(gp, wpc) * w
                pltpu.make_async_copy(
                    hsh.at[idxc.at[sel_p, pl.ds(offp, w)]],
                    rows.at[lax.rem(gp, 2)], sems.at[lax.rem(gp, 2)]).wait()

                def e_b(q, carry):
                    dvec = dstc[sel_p, pl.ds(offp + q * 16, 16)]
                    wvec = ewc[sel_p, pl.ds(offp + q * 16, 16)]
                    for i in range(16):
                        e = q * 16 + i
                        dl = dvec[i]
                        ww = wvec[i]
                        dlo = dl * 16
                        for cb in range(8):
                            sl = pl.ds(cb * 16, 16)
                            ds = pl.ds(dlo, 16)
                            accs[cb][ds] = jnp.maximum(
                                accs[cb][ds],
                                rows[lax.rem(gp, 2), e, sl] * ww)
                    return carry
                lax.fori_loop(0, w // 16, e_b, 0)
            return carry
        lax.fori_loop(0, nw + 1, step, 0)
        plsc.subcore_barrier()

    for part in range(4):
        def d_r(r, carry, part=part):
            for cb, acc in enumerate(accs):
                rows[0, r, pl.ds(cb * 16, 16)] = \
                    acc[pl.ds((part * 80 + r) * 16, 16)]
            return carry
        lax.fori_loop(0, 80, d_r, 0)
        pltpu.sync_copy(rows.at[0, pl.ds(0, 80)],
                        out_hbm.at[pl.ds(wid * RPT + part * 80, 80)])


def _segmax128(h, srcp, dstlp, ewp, nwin):
    f = pl.kernel(
        _segmax_body,
        out_type=jax.ShapeDtypeStruct((NPAD, 128), jnp.float32),
        mesh=_MESH,
        scratch_types=[
            *[pltpu.VMEM(((RPT + 1) * 16,), jnp.float32) for _ in range(8)],
            pltpu.VMEM((2, 2048), jnp.int32),
            pltpu.VMEM((2, 2048), jnp.int32),
            pltpu.VMEM((2, 2048), jnp.float32),
            pltpu.VMEM((2, 128, 128), jnp.float32),
            pltpu.VMEM((16,), jnp.int32),
            pltpu.VMEM_SHARED((HALF, 128), jnp.float32),
            pltpu.SemaphoreType.DMA((2,)),
        ],
    )
    return f(h, srcp, dstlp, ewp, nwin)


def _segmax(h, srcp, dstlp, ewp, nwin):
    c0 = h.shape[1]
    if c0 % 128:
        h = jnp.pad(h, ((0, 0), (0, 128 - c0 % 128)))
    c = h.shape[1]
    blocks = [_segmax128(h[:, i * 128:(i + 1) * 128], srcp, dstlp, ewp, nwin)
              for i in range(c // 128)]
    out = blocks[0] if len(blocks) == 1 else jnp.concatenate(blocks, axis=1)
    return out[:N, :c0]


def _mm_bias_relu_body(a_ref, w_ref, b_ref, o_ref):
    acc = jnp.dot(a_ref[...], w_ref[...], preferred_element_type=jnp.float32)
    o_ref[...] = jax.nn.relu(acc + b_ref[...])


def _mm_bias_relu(a, wt, b, bm=2000):
    m, k = a.shape
    _, o = wt.shape
    return pl.pallas_call(
        _mm_bias_relu_body,
        grid=(m // bm,),
        in_specs=[
            pl.BlockSpec((bm, k), lambda i: (i, 0)),
            pl.BlockSpec((k, o), lambda i: (0, 0)),
            pl.BlockSpec((1, o), lambda i: (0, 0)),
        ],
        out_specs=pl.BlockSpec((bm, o), lambda i: (i, 0)),
        out_shape=jax.ShapeDtypeStruct((m, o), jnp.float32),
    )(a, wt, b.reshape(1, -1))


def _mm2_bias_relu_body(a_ref, w1_ref, b_ref, n_ref, w2_ref, o_ref):
    acc = jnp.dot(a_ref[...], w1_ref[...], preferred_element_type=jnp.float32)
    acc += jnp.dot(n_ref[...], w2_ref[...], preferred_element_type=jnp.float32)
    o_ref[...] = jax.nn.relu(acc + b_ref[...])


def _mm2_bias_relu(a, w1t, b, neigh, w2t, bm=2000):
    m, k = a.shape
    _, o = w1t.shape
    return pl.pallas_call(
        _mm2_bias_relu_body,
        grid=(m // bm,),
        in_specs=[
            pl.BlockSpec((bm, k), lambda i: (i, 0)),
            pl.BlockSpec((k, o), lambda i: (0, 0)),
            pl.BlockSpec((1, o), lambda i: (0, 0)),
            pl.BlockSpec((bm, k), lambda i: (i, 0)),
            pl.BlockSpec((k, o), lambda i: (0, 0)),
        ],
        out_specs=pl.BlockSpec((bm, o), lambda i: (i, 0)),
        out_shape=jax.ShapeDtypeStruct((m, o), jnp.float32),
    )(a, w1t, b.reshape(1, -1), neigh, w2t)


def _adj_body(a_ref, b_ref, o_ref):
    o_ref[...] = jax.lax.dot_general(
        a_ref[...], b_ref[...], (((1,), (1,)), ((), ())),
        preferred_element_type=jnp.float32)


def _adj(hd, bm=2048):
    m, k = hd.shape
    return pl.pallas_call(
        _adj_body,
        grid=(pl.cdiv(m, bm), pl.cdiv(m, bm)),
        in_specs=[
            pl.BlockSpec((bm, k), lambda i, j: (i, 0)),
            pl.BlockSpec((bm, k), lambda i, j: (j, 0)),
        ],
        out_specs=pl.BlockSpec((bm, bm), lambda i, j: (i, j)),
        out_shape=jax.ShapeDtypeStruct((m, m), jnp.float32),
    )(hd, hd)


def kernel(feat, edge_weight, edge_index, enc1, enc2, enc3, dec1, dec2, dec3):
    src = edge_index[0]
    dst = edge_index[1]
    srcp, dstlp, ewp, nwin = _partition(src, dst, edge_weight)
    h = feat
    for params in (enc1, enc2, enc3, dec1, dec2, dec3):
        wp, bp, ws, bs, wn = params
        hp = _mm_bias_relu(h, wp.T, bp)
        neigh = _segmax(hp, srcp, dstlp, ewp, nwin)
        h = _mm2_bias_relu(h, ws.T, bs, neigh, wn.T)
    return (h, _adj(h))


# speedup vs baseline: 2.0350x; 2.0350x over previous
import functools

import jax
import jax.numpy as jnp
from jax import lax
from jax.experimental import pallas as pl
from jax.experimental.pallas import tpu as pltpu
from jax.experimental.pallas import tpu_sc as plsc

N = 10000
E = 320000
NC, NS = 2, 16
NW = NC * NS
RPT = 320
NPAD = NW * RPT
CAP = 16384
CAP2 = CAP // 2
HALF = N // 2
WF = 16000
W = 64

_MESH = plsc.VectorSubcoreMesh(core_axis_name="c", subcore_axis_name="s")


def _wid():
    return lax.axis_index("s") * NC + lax.axis_index("c")


def _partition_body(src_hbm, dst_hbm, ew_hbm, srcp, dstlp, ewp, nwin,
                    srcw, dstw, eww, srcl, dstl, ewl, nv):
    wid = _wid()
    lo = wid * RPT
    hi = jnp.minimum(lo + RPT, N)
    pad_src = lax.rem(lo, HALF)

    def init_b(i, carry):
        s = pl.ds(i * 16, 16)
        srcl[s] = jnp.full((16,), pad_src, jnp.int32)
        dstl[s] = jnp.full((16,), RPT, jnp.int32)
        ewl[s] = jnp.zeros((16,), jnp.float32)
        return carry
    lax.fori_loop(0, CAP // 16, init_b, 0)

    def win_b(g, ptrs):
        base = g * WF
        pltpu.sync_copy(src_hbm.at[pl.ds(base, WF)], srcw)
        pltpu.sync_copy(dst_hbm.at[pl.ds(base, WF)], dstw)
        pltpu.sync_copy(ew_hbm.at[pl.ds(base, WF)], eww)

        def vec_b(j, ptrs):
            p0, p1 = ptrs
            s = pl.ds(j * 16, 16)
            dv = dstw[s]
            sv = srcw[s]
            wv = eww[s]
            m = (dv >= lo) & (dv < hi)
            lane = lax.iota(jnp.int32, 16)
            outp = []
            for mk, pk, rbase, shift in ((m & (sv < HALF), p0, 0, 0),
                                         (m & (sv >= HALF), p1, CAP2, HALF)):
                x = jnp.where(mk, 1, 0)
                for k in (1, 2, 4, 8):
                    sh = x.at[jnp.maximum(lane - k, 0)].get(
                        mode="promise_in_bounds")
                    x = x + jnp.where(lane >= k, sh, 0)
                cnt = x[15]
                tgt = lane + 1
                b = jnp.zeros((16,), jnp.int32)
                for k in (8, 4, 2, 1):
                    pv = x.at[b + (k - 1)].get(mode="promise_in_bounds")
                    b = b + jnp.where(pv < tgt, k, 0)
                perm = jnp.minimum(b, 15)
                pc = jnp.minimum(pk, CAP2 - 16)

                def tk(v, perm=perm):
                    return v.at[perm].get(mode="promise_in_bounds")
                dstl[pl.ds(rbase + pc, 16)] = tk(dv - lo)
                srcl[pl.ds(rbase + pc, 16)] = tk(sv - shift)
                ewl[pl.ds(rbase + pc, 16)] = tk(wv)
                outp.append(pc + cnt)
            return tuple(outp)
        return lax.fori_loop(0, WF // 16, vec_b, ptrs)

    cnt0, cnt1 = lax.fori_loop(0, E // WF, win_b,
                               (jnp.int32(0), jnp.int32(0)))
    for ck, rbase in ((cnt0, 0), (cnt1, CAP2)):
        pt = rbase + jnp.minimum(ck, CAP2 - 16)
        srcl[pl.ds(pt, 16)] = jnp.full((16,), pad_src, jnp.int32)
        dstl[pl.ds(pt, 16)] = jnp.full((16,), RPT, jnp.int32)
        ewl[pl.ds(pt, 16)] = jnp.zeros((16,), jnp.float32)
    lane = lax.iota(jnp.int32, 16)
    nv[...] = jnp.where(lane < 8, cnt0, cnt1)
    pltpu.sync_copy(nv, nwin.at[pl.ds(wid * 16, 16)])
    pltpu.sync_copy(srcl.at[pl.ds(0, CAP)], srcp.at[pl.ds(wid * CAP, CAP)])
    pltpu.sync_copy(dstl.at[pl.ds(0, CAP)], dstlp.at[pl.ds(wid * CAP, CAP)])
    pltpu.sync_copy(ewl.at[pl.ds(0, CAP)], ewp.at[pl.ds(wid * CAP, CAP)])


def _partition(src, dst, ew):
    f = pl.kernel(
        _partition_body,
        out_type=[
            jax.ShapeDtypeStruct((NW * CAP,), jnp.int32),
            jax.ShapeDtypeStruct((NW * CAP,), jnp.int32),
            jax.ShapeDtypeStruct((NW * CAP,), jnp.float32),
            jax.ShapeDtypeStruct((NW * 16,), jnp.int32),
        ],
        mesh=_MESH,
        scratch_types=[
            pltpu.VMEM((WF,), jnp.int32),
            pltpu.VMEM((WF,), jnp.int32),
            pltpu.VMEM((WF,), jnp.float32),
            pltpu.VMEM((CAP + 16,), jnp.int32),
            pltpu.VMEM((CAP + 16,), jnp.int32),
            pltpu.VMEM((CAP + 16,), jnp.float32),
            pltpu.VMEM((16,), jnp.int32),
        ],
    )
    return f(src, dst, ew)


def _segmax_body(h_hbm, srcp, dstlp, ewp, nwin, out_hbm,
                 a0, a1, a2, a3, a4, a5, a6, a7,
                 idxc, dstc, ewc, rows, nv, hsh, sems):
    wid = _wid()
    sid = lax.axis_index("s")
    w = 128
    chunk = 2048
    wpc = chunk // w
    accs = (a0, a1, a2, a3, a4, a5, a6, a7)

    def z_r(r, carry):
        for acc in accs:
            acc[pl.ds(r * 16, 16)] = jnp.zeros((16,), jnp.float32)
        return carry
    lax.fori_loop(0, RPT + 1, z_r, 0)

    pltpu.sync_copy(nwin.at[pl.ds(wid * 16, 16)], nv)
    nvv = nv[...]

    for half, lane0 in ((0, 0), (1, 8)):
        nw = lax.div(nvv[lane0] + (w - 1), w)
        lbase = wid * CAP + half * CAP2
        hrow = half * HALF

        pltpu.sync_copy(h_hbm.at[pl.ds(hrow + sid * 312, 312)],
                        hsh.at[pl.ds(sid * 312, 312)])

        @pl.when(sid == 0)
        def _tail(hrow=hrow):
            pltpu.sync_copy(h_hbm.at[pl.ds(hrow + 4992, 8)],
                            hsh.at[pl.ds(4992, 8)])
        plsc.subcore_barrier()

        def step(g, carry, nw=nw, lbase=lbase):
            sel_c = lax.rem(lax.div(g, wpc), 2)
            off = lax.rem(g, wpc) * w

            @pl.when(g < nw)
            def _issue():
                @pl.when(lax.rem(g, wpc) == 0)
                def _load_chunk():
                    cb = lbase + lax.div(g, wpc) * chunk
                    pltpu.sync_copy(srcp.at[pl.ds(cb, chunk)],
                                    idxc.at[sel_c])
                    pltpu.sync_copy(dstlp.at[pl.ds(cb, chunk)],
                                    dstc.at[sel_c])
                    pltpu.sync_copy(ewp.at[pl.ds(cb, chunk)], ewc.at[sel_c])
                pltpu.async_copy(hsh.at[idxc.at[sel_c, pl.ds(off, w)]],
                                 rows.at[lax.rem(g, 2)],
                                 sems.at[lax.rem(g, 2)])

            @pl.when(g > 0)
            def _compute():
                gp = g - 1
                sel_p = lax.rem(lax.div(gp, wpc), 2)
                offp = lax.rem(gp, wpc) * w
                pltpu.make_async_copy(
                    hsh.at[idxc.at[sel_p, pl.ds(offp, w)]],
                    rows.at[lax.rem(gp, 2)], sems.at[lax.rem(gp, 2)]).wait()

                def e_b(q, carry):
                    dvec = dstc[sel_p, pl.ds(offp + q * 16, 16)]
                    wvec = ewc[sel_p, pl.ds(offp + q * 16, 16)]
                    for i in range(16):
                        e = q * 16 + i
                        dl = dvec[i]
                        ww = wvec[i]
                        dlo = dl * 16
                        mvals = [rows[lax.rem(gp, 2), e, pl.ds(cb * 16, 16)]
                                 * ww for cb in range(8)]
                        for cb in range(8):
                            ds = pl.ds(dlo, 16)
                            accs[cb][ds] = jnp.maximum(accs[cb][ds],
                                                       mvals[cb])
                    return carry
                lax.fori_loop(0, w // 16, e_b, 0)
            return carry
        lax.fori_loop(0, nw + 1, step, 0)
        plsc.subcore_barrier()

    for part in range(4):
        def d_r(r, carry, part=part):
            for cb, acc in enumerate(accs):
                rows[0, r, pl.ds(cb * 16, 16)] = \
                    acc[pl.ds((part * 80 + r) * 16, 16)]
            return carry
        lax.fori_loop(0, 80, d_r, 0)
        pltpu.sync_copy(rows.at[0, pl.ds(0, 80)],
                        out_hbm.at[pl.ds(wid * RPT + part * 80, 80)])


def _segmax128(h, srcp, dstlp, ewp, nwin):
    f = pl.kernel(
        _segmax_body,
        out_type=jax.ShapeDtypeStruct((NPAD, 128), jnp.float32),
        mesh=_MESH,
        scratch_types=[
            *[pltpu.VMEM(((RPT + 1) * 16,), jnp.float32) for _ in range(8)],
            pltpu.VMEM((2, 2048), jnp.int32),
            pltpu.VMEM((2, 2048), jnp.int32),
            pltpu.VMEM((2, 2048), jnp.float32),
            pltpu.VMEM((2, 128, 128), jnp.float32),
            pltpu.VMEM((16,), jnp.int32),
            pltpu.VMEM_SHARED((HALF, 128), jnp.float32),
            pltpu.SemaphoreType.DMA((2,)),
        ],
    )
    return f(h, srcp, dstlp, ewp, nwin)


def _segmax(h, srcp, dstlp, ewp, nwin):
    c0 = h.shape[1]
    if c0 % 128:
        h = jnp.pad(h, ((0, 0), (0, 128 - c0 % 128)))
    c = h.shape[1]
    blocks = [_segmax128(h[:, i * 128:(i + 1) * 128], srcp, dstlp, ewp, nwin)
              for i in range(c // 128)]
    out = blocks[0] if len(blocks) == 1 else jnp.concatenate(blocks, axis=1)
    return out[:N, :c0]


def _mm_bias_relu_body(a_ref, w_ref, b_ref, o_ref):
    acc = jnp.dot(a_ref[...], w_ref[...], preferred_element_type=jnp.float32)
    o_ref[...] = jax.nn.relu(acc + b_ref[...])


def _mm_bias_relu(a, wt, b, bm=2000):
    m, k = a.shape
    _, o = wt.shape
    return pl.pallas_call(
        _mm_bias_relu_body,
        grid=(m // bm,),
        in_specs=[
            pl.BlockSpec((bm, k), lambda i: (i, 0)),
            pl.BlockSpec((k, o), lambda i: (0, 0)),
            pl.BlockSpec((1, o), lambda i: (0, 0)),
        ],
        out_specs=pl.BlockSpec((bm, o), lambda i: (i, 0)),
        out_shape=jax.ShapeDtypeStruct((m, o), jnp.float32),
    )(a, wt, b.reshape(1, -1))


def _mm2_bias_relu_body(a_ref, w1_ref, b_ref, n_ref, w2_ref, o_ref):
    acc = jnp.dot(a_ref[...], w1_ref[...], preferred_element_type=jnp.float32)
    acc += jnp.dot(n_ref[...], w2_ref[...], preferred_element_type=jnp.float32)
    o_ref[...] = jax.nn.relu(acc + b_ref[...])


def _mm2_bias_relu(a, w1t, b, neigh, w2t, bm=2000):
    m, k = a.shape
    _, o = w1t.shape
    return pl.pallas_call(
        _mm2_bias_relu_body,
        grid=(m // bm,),
        in_specs=[
            pl.BlockSpec((bm, k), lambda i: (i, 0)),
            pl.BlockSpec((k, o), lambda i: (0, 0)),
            pl.BlockSpec((1, o), lambda i: (0, 0)),
            pl.BlockSpec((bm, k), lambda i: (i, 0)),
            pl.BlockSpec((k, o), lambda i: (0, 0)),
        ],
        out_specs=pl.BlockSpec((bm, o), lambda i: (i, 0)),
        out_shape=jax.ShapeDtypeStruct((m, o), jnp.float32),
    )(a, w1t, b.reshape(1, -1), neigh, w2t)


def _adj_body(a_ref, b_ref, o_ref):
    o_ref[...] = jax.lax.dot_general(
        a_ref[...], b_ref[...], (((1,), (1,)), ((), ())),
        preferred_element_type=jnp.float32)


def _adj(hd, bm=2048):
    m, k = hd.shape
    return pl.pallas_call(
        _adj_body,
        grid=(pl.cdiv(m, bm), pl.cdiv(m, bm)),
        in_specs=[
            pl.BlockSpec((bm, k), lambda i, j: (i, 0)),
            pl.BlockSpec((bm, k), lambda i, j: (j, 0)),
        ],
        out_specs=pl.BlockSpec((bm, bm), lambda i, j: (i, j)),
        out_shape=jax.ShapeDtypeStruct((m, m), jnp.float32),
    )(hd, hd)


def kernel(feat, edge_weight, edge_index, enc1, enc2, enc3, dec1, dec2, dec3):
    src = edge_index[0]
    dst = edge_index[1]
    srcp, dstlp, ewp, nwin = _partition(src, dst, edge_weight)
    h = feat
    for params in (enc1, enc2, enc3, dec1, dec2, dec3):
        wp, bp, ws, bs, wn = params
        hp = _mm_bias_relu(h, wp.T, bp)
        neigh = _segmax(hp, srcp, dstlp, ewp, nwin)
        h = _mm2_bias_relu(h, ws.T, bs, neigh, wn.T)
    return (h, _adj(h))


# speedup vs baseline: 2.0371x; 1.0011x over previous
import functools

import jax
import jax.numpy as jnp
from jax import lax
from jax.experimental import pallas as pl
from jax.experimental.pallas import tpu as pltpu
from jax.experimental.pallas import tpu_sc as plsc

N = 10000
E = 320000
NC, NS = 2, 16
NW = NC * NS
RPT = 320
NPAD = NW * RPT
CAP = 16384
CAP2 = CAP // 2
HALF = N // 2
WF = 16000
W = 64

_MESH = plsc.VectorSubcoreMesh(core_axis_name="c", subcore_axis_name="s")


def _wid():
    return lax.axis_index("s") * NC + lax.axis_index("c")


def _partition_body(src_hbm, dst_hbm, ew_hbm, srcp, dstlp, ewp, nwin,
                    srcw, dstw, eww, srcl, dstl, ewl, nv):
    wid = _wid()
    lo = wid * RPT
    hi = jnp.minimum(lo + RPT, N)
    pad_src = lax.rem(lo, HALF)

    def init_b(i, carry):
        s = pl.ds(i * 16, 16)
        srcl[s] = jnp.full((16,), pad_src, jnp.int32)
        dstl[s] = jnp.full((16,), RPT, jnp.int32)
        ewl[s] = jnp.zeros((16,), jnp.float32)
        return carry
    lax.fori_loop(0, CAP // 16, init_b, 0)

    def win_b(g, ptrs):
        base = g * WF
        pltpu.sync_copy(src_hbm.at[pl.ds(base, WF)], srcw)
        pltpu.sync_copy(dst_hbm.at[pl.ds(base, WF)], dstw)
        pltpu.sync_copy(ew_hbm.at[pl.ds(base, WF)], eww)

        def vec_b(j, ptrs):
            p0, p1 = ptrs
            s = pl.ds(j * 16, 16)
            dv = dstw[s]
            sv = srcw[s]
            wv = eww[s]
            m = (dv >= lo) & (dv < hi)
            lane = lax.iota(jnp.int32, 16)
            ms = (m & (sv < HALF), m & (sv >= HALF))
            xs = [jnp.where(mk, 1, 0) for mk in ms]
            for k in (1, 2, 4, 8):
                shs = [x.at[jnp.maximum(lane - k, 0)].get(
                    mode="promise_in_bounds") for x in xs]
                xs = [x + jnp.where(lane >= k, sh, 0)
                      for x, sh in zip(xs, shs)]
            cnts = [x[15] for x in xs]
            tgt = lane + 1
            bs = [jnp.zeros((16,), jnp.int32)] * 2
            for k in (8, 4, 2, 1):
                pvs = [x.at[b + (k - 1)].get(mode="promise_in_bounds")
                       for x, b in zip(xs, bs)]
                bs = [b + jnp.where(pv < tgt, k, 0)
                      for b, pv in zip(bs, pvs)]
            perms = [jnp.minimum(b, 15) for b in bs]
            pcs = [jnp.minimum(p0, CAP2 - 16), jnp.minimum(p1, CAP2 - 16)]

            def tk(v, perm):
                return v.at[perm].get(mode="promise_in_bounds")
            vals = [[tk(dv - lo, pm), tk(sv - sh, pm), tk(wv, pm)]
                    for pm, sh in zip(perms, (0, HALF))]
            for vv, pc, rbase in zip(vals, pcs, (0, CAP2)):
                dstl[pl.ds(rbase + pc, 16)] = vv[0]
                srcl[pl.ds(rbase + pc, 16)] = vv[1]
                ewl[pl.ds(rbase + pc, 16)] = vv[2]
            return (pcs[0] + cnts[0], pcs[1] + cnts[1])
        return lax.fori_loop(0, WF // 16, vec_b, ptrs)

    cnt0, cnt1 = lax.fori_loop(0, E // WF, win_b,
                               (jnp.int32(0), jnp.int32(0)))
    for ck, rbase in ((cnt0, 0), (cnt1, CAP2)):
        pt = rbase + jnp.minimum(ck, CAP2 - 16)
        srcl[pl.ds(pt, 16)] = jnp.full((16,), pad_src, jnp.int32)
        dstl[pl.ds(pt, 16)] = jnp.full((16,), RPT, jnp.int32)
        ewl[pl.ds(pt, 16)] = jnp.zeros((16,), jnp.float32)
    lane = lax.iota(jnp.int32, 16)
    nv[...] = jnp.where(lane < 8, cnt0, cnt1)
    pltpu.sync_copy(nv, nwin.at[pl.ds(wid * 16, 16)])
    pltpu.sync_copy(srcl.at[pl.ds(0, CAP)], srcp.at[pl.ds(wid * CAP, CAP)])
    pltpu.sync_copy(dstl.at[pl.ds(0, CAP)], dstlp.at[pl.ds(wid * CAP, CAP)])
    pltpu.sync_copy(ewl.at[pl.ds(0, CAP)], ewp.at[pl.ds(wid * CAP, CAP)])


def _partition(src, dst, ew):
    f = pl.kernel(
        _partition_body,
        out_type=[
            jax.ShapeDtypeStruct((NW * CAP,), jnp.int32),
            jax.ShapeDtypeStruct((NW * CAP,), jnp.int32),
            jax.ShapeDtypeStruct((NW * CAP,), jnp.float32),
            jax.ShapeDtypeStruct((NW * 16,), jnp.int32),
        ],
        mesh=_MESH,
        scratch_types=[
            pltpu.VMEM((WF,), jnp.int32),
            pltpu.VMEM((WF,), jnp.int32),
            pltpu.VMEM((WF,), jnp.float32),
            pltpu.VMEM((CAP + 16,), jnp.int32),
            pltpu.VMEM((CAP + 16,), jnp.int32),
            pltpu.VMEM((CAP + 16,), jnp.float32),
            pltpu.VMEM((16,), jnp.int32),
        ],
    )
    return f(src, dst, ew)


def _segmax_body(h_hbm, srcp, dstlp, ewp, nwin, out_hbm,
                 a0, a1, a2, a3, a4, a5, a6, a7,
                 idxc, dstc, ewc, rows, nv, hsh, sems):
    wid = _wid()
    sid = lax.axis_index("s")
    w = 128
    chunk = 2048
    wpc = chunk // w
    accs = (a0, a1, a2, a3, a4, a5, a6, a7)

    def z_r(r, carry):
        for acc in accs:
            acc[pl.ds(r * 16, 16)] = jnp.zeros((16,), jnp.float32)
        return carry
    lax.fori_loop(0, RPT + 1, z_r, 0)

    pltpu.sync_copy(nwin.at[pl.ds(wid * 16, 16)], nv)
    nvv = nv[...]

    for half, lane0 in ((0, 0), (1, 8)):
        nw = lax.div(nvv[lane0] + (w - 1), w)
        lbase = wid * CAP + half * CAP2
        hrow = half * HALF

        pltpu.sync_copy(h_hbm.at[pl.ds(hrow + sid * 312, 312)],
                        hsh.at[pl.ds(sid * 312, 312)])

        @pl.when(sid == 0)
        def _tail(hrow=hrow):
            pltpu.sync_copy(h_hbm.at[pl.ds(hrow + 4992, 8)],
                            hsh.at[pl.ds(4992, 8)])
        plsc.subcore_barrier()

        def step(g, carry, nw=nw, lbase=lbase):
            sel_c = lax.rem(lax.div(g, wpc), 2)
            off = lax.rem(g, wpc) * w

            @pl.when(g < nw)
            def _issue():
                @pl.when(lax.rem(g, wpc) == 0)
                def _load_chunk():
                    cb = lbase + lax.div(g, wpc) * chunk
                    pltpu.sync_copy(srcp.at[pl.ds(cb, chunk)],
                                    idxc.at[sel_c])
                    pltpu.sync_copy(dstlp.at[pl.ds(cb, chunk)],
                                    dstc.at[sel_c])
                    pltpu.sync_copy(ewp.at[pl.ds(cb, chunk)], ewc.at[sel_c])
                pltpu.async_copy(hsh.at[idxc.at[sel_c, pl.ds(off, w)]],
                                 rows.at[lax.rem(g, 2)],
                                 sems.at[lax.rem(g, 2)])

            @pl.when(g > 0)
            def _compute():
                gp = g - 1
                sel_p = lax.rem(lax.div(gp, wpc), 2)
                offp = lax.rem(gp, wpc) * w
                pltpu.make_async_copy(
                    hsh.at[idxc.at[sel_p, pl.ds(offp, w)]],
                    rows.at[lax.rem(gp, 2)], sems.at[lax.rem(gp, 2)]).wait()

                def e_b(q, carry):
                    dvec = dstc[sel_p, pl.ds(offp + q * 16, 16)]
                    wvec = ewc[sel_p, pl.ds(offp + q * 16, 16)]
                    for i in range(16):
                        e = q * 16 + i
                        dl = dvec[i]
                        ww = wvec[i]
                        dlo = dl * 16
                        mvals = [rows[lax.rem(gp, 2), e, pl.ds(cb * 16, 16)]
                                 * ww for cb in range(8)]
                        for cb in range(8):
                            ds = pl.ds(dlo, 16)
                            accs[cb][ds] = jnp.maximum(accs[cb][ds],
                                                       mvals[cb])
                    return carry
                lax.fori_loop(0, w // 16, e_b, 0)
            return carry
        lax.fori_loop(0, nw + 1, step, 0)
        plsc.subcore_barrier()

    for part in range(4):
        def d_r(r, carry, part=part):
            for cb, acc in enumerate(accs):
                rows[0, r, pl.ds(cb * 16, 16)] = \
                    acc[pl.ds((part * 80 + r) * 16, 16)]
            return carry
        lax.fori_loop(0, 80, d_r, 0)
        pltpu.sync_copy(rows.at[0, pl.ds(0, 80)],
                        out_hbm.at[pl.ds(wid * RPT + part * 80, 80)])


def _segmax128(h, srcp, dstlp, ewp, nwin):
    f = pl.kernel(
        _segmax_body,
        out_type=jax.ShapeDtypeStruct((NPAD, 128), jnp.float32),
        mesh=_MESH,
        scratch_types=[
            *[pltpu.VMEM(((RPT + 1) * 16,), jnp.float32) for _ in range(8)],
            pltpu.VMEM((2, 2048), jnp.int32),
            pltpu.VMEM((2, 2048), jnp.int32),
            pltpu.VMEM((2, 2048), jnp.float32),
            pltpu.VMEM((2, 128, 128), jnp.float32),
            pltpu.VMEM((16,), jnp.int32),
            pltpu.VMEM_SHARED((HALF, 128), jnp.float32),
            pltpu.SemaphoreType.DMA((2,)),
        ],
    )
    return f(h, srcp, dstlp, ewp, nwin)


def _segmax(h, srcp, dstlp, ewp, nwin):
    c0 = h.shape[1]
    if c0 % 128:
        h = jnp.pad(h, ((0, 0), (0, 128 - c0 % 128)))
    c = h.shape[1]
    blocks = [_segmax128(h[:, i * 128:(i + 1) * 128], srcp, dstlp, ewp, nwin)
              for i in range(c // 128)]
    out = blocks[0] if len(blocks) == 1 else jnp.concatenate(blocks, axis=1)
    return out[:N, :c0]


def _mm_bias_relu_body(a_ref, w_ref, b_ref, o_ref):
    acc = jnp.dot(a_ref[...], w_ref[...], preferred_element_type=jnp.float32)
    o_ref[...] = jax.nn.relu(acc + b_ref[...])


def _mm_bias_relu(a, wt, b, bm=2000):
    m, k = a.shape
    _, o = wt.shape
    return pl.pallas_call(
        _mm_bias_relu_body,
        grid=(m // bm,),
        in_specs=[
            pl.BlockSpec((bm, k), lambda i: (i, 0)),
            pl.BlockSpec((k, o), lambda i: (0, 0)),
            pl.BlockSpec((1, o), lambda i: (0, 0)),
        ],
        out_specs=pl.BlockSpec((bm, o), lambda i: (i, 0)),
        out_shape=jax.ShapeDtypeStruct((m, o), jnp.float32),
    )(a, wt, b.reshape(1, -1))


def _mm2_bias_relu_body(a_ref, w1_ref, b_ref, n_ref, w2_ref, o_ref):
    acc = jnp.dot(a_ref[...], w1_ref[...], preferred_element_type=jnp.float32)
    acc += jnp.dot(n_ref[...], w2_ref[...], preferred_element_type=jnp.float32)
    o_ref[...] = jax.nn.relu(acc + b_ref[...])


def _mm2_bias_relu(a, w1t, b, neigh, w2t, bm=2000):
    m, k = a.shape
    _, o = w1t.shape
    return pl.pallas_call(
        _mm2_bias_relu_body,
        grid=(m // bm,),
        in_specs=[
            pl.BlockSpec((bm, k), lambda i: (i, 0)),
            pl.BlockSpec((k, o), lambda i: (0, 0)),
            pl.BlockSpec((1, o), lambda i: (0, 0)),
            pl.BlockSpec((bm, k), lambda i: (i, 0)),
            pl.BlockSpec((k, o), lambda i: (0, 0)),
        ],
        out_specs=pl.BlockSpec((bm, o), lambda i: (i, 0)),
        out_shape=jax.ShapeDtypeStruct((m, o), jnp.float32),
    )(a, w1t, b.reshape(1, -1), neigh, w2t)


def _adj_body(a_ref, b_ref, o_ref):
    o_ref[...] = jax.lax.dot_general(
        a_ref[...], b_ref[...], (((1,), (1,)), ((), ())),
        preferred_element_type=jnp.float32)


def _adj(hd, bm=2048):
    m, k = hd.shape
    return pl.pallas_call(
        _adj_body,
        grid=(pl.cdiv(m, bm), pl.cdiv(m, bm)),
        in_specs=[
            pl.BlockSpec((bm, k), lambda i, j: (i, 0)),
            pl.BlockSpec((bm, k), lambda i, j: (j, 0)),
        ],
        out_specs=pl.BlockSpec((bm, bm), lambda i, j: (i, j)),
        out_shape=jax.ShapeDtypeStruct((m, m), jnp.float32),
    )(hd, hd)


def kernel(feat, edge_weight, edge_index, enc1, enc2, enc3, dec1, dec2, dec3):
    src = edge_index[0]
    dst = edge_index[1]
    srcp, dstlp, ewp, nwin = _partition(src, dst, edge_weight)
    h = feat
    for params in (enc1, enc2, enc3, dec1, dec2, dec3):
        wp, bp, ws, bs, wn = params
        hp = _mm_bias_relu(h, wp.T, bp)
        neigh = _segmax(hp, srcp, dstlp, ewp, nwin)
        h = _mm2_bias_relu(h, ws.T, bs, neigh, wn.T)
    return (h, _adj(h))
